# baseline (speedup 1.0000x reference)
import functools

import jax
import jax.numpy as jnp
from jax import lax
from jax.experimental import pallas as pl
from jax.experimental.pallas import tpu as pltpu
from jax.experimental.pallas import tpu_sc as plsc


def _gather_rows_sc(table, idx):
    batch = idx.shape[0]
    embed = table.shape[1]
    info = plsc.get_sparse_core_info()
    nc, ns = info.num_cores, info.num_subcores
    nw = nc * ns
    b_per_w = batch // nw
    mesh = plsc.VectorSubcoreMesh(core_axis_name="c", subcore_axis_name="s")

    @functools.partial(
        pl.kernel,
        mesh=mesh,
        compiler_params=pltpu.CompilerParams(use_tc_tiling_on_sc=False),
        out_type=jax.ShapeDtypeStruct((batch, embed), jnp.float32),
        scratch_types=[
            pltpu.VMEM((b_per_w,), jnp.int32),
            pltpu.VMEM((b_per_w, embed), jnp.float32),
            pltpu.SemaphoreType.DMA,
        ],
    )
    def gather_kernel(table_hbm, idx_hbm, out_hbm, idx_v, rows_v, sem):
        wid = lax.axis_index("s") * nc + lax.axis_index("c")
        base = wid * b_per_w
        pltpu.sync_copy(idx_hbm.at[pl.ds(base, b_per_w)], idx_v)
        pltpu.async_copy(table_hbm.at[idx_v], rows_v, sem).wait()
        pltpu.sync_copy(rows_v, out_hbm.at[pl.ds(base, b_per_w)])

    return gather_kernel(table, idx)


_BLOCK_N = 2048


def _score_tc(users_emb, items):
    m, k = users_emb.shape
    n = items.shape[0]

    def body(u_ref, it_ref, o_ref):
        s = lax.dot_general(
            u_ref[...], it_ref[...], (((1,), (1,)), ((), ())),
            preferred_element_type=jnp.float32,
        )
        o_ref[...] = jax.nn.sigmoid(s)

    return pl.pallas_call(
        body,
        grid=(pl.cdiv(n, _BLOCK_N),),
        in_specs=[
            pl.BlockSpec((m, k), lambda j: (0, 0)),
            pl.BlockSpec((_BLOCK_N, k), lambda j: (j, 0)),
        ],
        out_specs=pl.BlockSpec((m, _BLOCK_N), lambda j: (0, j)),
        out_shape=jax.ShapeDtypeStruct((m, n), jnp.float32),
    )(users_emb, items)


def kernel(users, embedding_user_weight, embedding_item_weight):
    idx = users.astype(jnp.int32)
    users_emb = _gather_rows_sc(embedding_user_weight, idx)
    return _score_tc(users_emb, embedding_item_weight)


# speedup vs baseline: 1.0063x; 1.0063x over previous
import functools

import jax
import jax.numpy as jnp
from jax import lax
from jax.experimental import pallas as pl
from jax.experimental.pallas import tpu as pltpu
from jax.experimental.pallas import tpu_sc as plsc


def _gather_rows_sc(table, idx):
    batch = idx.shape[0]
    embed = table.shape[1]
    info = plsc.get_sparse_core_info()
    nc, ns = info.num_cores, info.num_subcores
    nw = nc * ns
    b_per_w = batch // nw
    mesh = plsc.VectorSubcoreMesh(core_axis_name="c", subcore_axis_name="s")

    @functools.partial(
        pl.kernel,
        mesh=mesh,
        compiler_params=pltpu.CompilerParams(use_tc_tiling_on_sc=False),
        out_type=jax.ShapeDtypeStruct((batch, embed), jnp.float32),
        scratch_types=[
            pltpu.VMEM((b_per_w,), jnp.int32),
            pltpu.VMEM((b_per_w, embed), jnp.float32),
            pltpu.SemaphoreType.DMA,
        ],
    )
    def gather_kernel(table_hbm, idx_hbm, out_hbm, idx_v, rows_v, sem):
        wid = lax.axis_index("s") * nc + lax.axis_index("c")
        base = wid * b_per_w
        pltpu.sync_copy(idx_hbm.at[pl.ds(base, b_per_w)], idx_v)
        pltpu.async_copy(table_hbm.at[idx_v], rows_v, sem).wait()
        pltpu.sync_copy(rows_v, out_hbm.at[pl.ds(base, b_per_w)])

    return gather_kernel(table, idx)


_BLOCK_N = 2048


def _score_tc(users_emb, items):
    m, k = users_emb.shape
    n = items.shape[0]

    def body(u_ref, it_ref, o_ref):
        s = lax.dot_general(
            u_ref[...], it_ref[...], (((1,), (1,)), ((), ())),
            preferred_element_type=jnp.float32,
        )
        o_ref[...] = 0.5 * jnp.tanh(0.5 * s) + 0.5

    return pl.pallas_call(
        body,
        grid=(pl.cdiv(n, _BLOCK_N),),
        in_specs=[
            pl.BlockSpec((m, k), lambda j: (0, 0)),
            pl.BlockSpec((_BLOCK_N, k), lambda j: (j, 0)),
        ],
        out_specs=pl.BlockSpec((m, _BLOCK_N), lambda j: (0, j)),
        out_shape=jax.ShapeDtypeStruct((m, n), jnp.float32),
    )(users_emb, items)


def kernel(users, embedding_user_weight, embedding_item_weight):
    idx = users.astype(jnp.int32)
    users_emb = _gather_rows_sc(embedding_user_weight, idx)
    return _score_tc(users_emb, embedding_item_weight)


# speedup vs baseline: 1.3136x; 1.3054x over previous
import functools

import jax
import jax.numpy as jnp
from jax import lax
from jax.experimental import pallas as pl
from jax.experimental.pallas import tpu as pltpu
from jax.experimental.pallas import tpu_sc as plsc


def _gather_rows_sc(table, idx):
    batch = idx.shape[0]
    embed = table.shape[1]
    info = plsc.get_sparse_core_info()
    nc, ns = info.num_cores, info.num_subcores
    nw = nc * ns
    b_per_w = batch // nw
    mesh = plsc.VectorSubcoreMesh(core_axis_name="c", subcore_axis_name="s")

    @functools.partial(
        pl.kernel,
        mesh=mesh,
        out_type=jax.ShapeDtypeStruct((batch, embed), jnp.float32),
        scratch_types=[
            pltpu.VMEM((b_per_w,), jnp.int32),
            pltpu.VMEM((b_per_w, embed), jnp.float32),
            pltpu.SemaphoreType.DMA,
            pltpu.SemaphoreType.DMA,
        ],
    )
    def gather_kernel(table_hbm, idx_hbm, out_hbm, idx_v, rows_v, sem, osem):
        wid = lax.axis_index("s") * nc + lax.axis_index("c")
        base = wid * b_per_w
        pltpu.sync_copy(idx_hbm.at[pl.ds(base, b_per_w)], idx_v)
        copies = []
        for g in range(b_per_w // 16):
            vec = idx_v[pl.ds(g * 16, 16)]
            for j in range(16):
                r = g * 16 + j
                copies.append(
                    pltpu.async_copy(table_hbm.at[vec[j]], rows_v.at[r], sem))
        for c in copies:
            c.wait()
        pltpu.async_copy(rows_v, out_hbm.at[pl.ds(base, b_per_w)], osem).wait()

    return gather_kernel(table, idx)


_BLOCK_N = 2048


def _score_tc(users_emb, items):
    m, k = users_emb.shape
    n = items.shape[0]

    def body(u_ref, it_ref, o_ref):
        s = lax.dot_general(
            u_ref[...], it_ref[...], (((1,), (1,)), ((), ())),
            preferred_element_type=jnp.float32,
        )
        o_ref[...] = 0.5 * jnp.tanh(0.5 * s) + 0.5

    return pl.pallas_call(
        body,
        grid=(pl.cdiv(n, _BLOCK_N),),
        in_specs=[
            pl.BlockSpec((m, k), lambda j: (0, 0)),
            pl.BlockSpec((_BLOCK_N, k), lambda j: (j, 0)),
        ],
        out_specs=pl.BlockSpec((m, _BLOCK_N), lambda j: (0, j)),
        out_shape=jax.ShapeDtypeStruct((m, n), jnp.float32),
    )(users_emb, items)


def kernel(users, embedding_user_weight, embedding_item_weight):
    idx = users.astype(jnp.int32)
    users_emb = _gather_rows_sc(embedding_user_weight, idx)
    return _score_tc(users_emb, embedding_item_weight)


# speedup vs baseline: 6.1740x; 4.7001x over previous
import functools

import jax
import jax.numpy as jnp
from jax import lax
from jax.experimental import pallas as pl
from jax.experimental.pallas import tpu as pltpu
from jax.experimental.pallas import tpu_sc as plsc

_LANES = 128


def _gather_rows_sc(table_t, idx):
    embed, _ = table_t.shape
    batch = idx.shape[0]
    info = plsc.get_sparse_core_info()
    nc, ns, nl = info.num_cores, info.num_subcores, info.num_lanes
    nw = nc * ns
    b_per_w = batch // nw
    mesh = plsc.VectorSubcoreMesh(core_axis_name="c", subcore_axis_name="s")

    @functools.partial(
        pl.kernel,
        mesh=mesh,
        compiler_params=pltpu.CompilerParams(needs_layout_passes=False),
        out_type=jax.ShapeDtypeStruct((batch, embed), jnp.float32),
        scratch_types=[
            pltpu.VMEM((b_per_w,), jnp.int32),
            pltpu.VMEM((2, embed, _LANES), jnp.float32),
            pltpu.VMEM((b_per_w, embed), jnp.float32),
            pltpu.SemaphoreType.DMA((2,)),
            pltpu.SemaphoreType.DMA,
        ],
    )
    def gather_kernel(table_hbm, idx_hbm, out_hbm, idx_v, slab_v, rows_v,
                      sems, osem):
        wid = lax.axis_index("s") * nc + lax.axis_index("c")
        base = wid * b_per_w
        pltpu.sync_copy(idx_hbm.at[pl.ds(base, b_per_w)], idx_v)
        lanes, starts = [], []
        for g in range(b_per_w // nl):
            vec = idx_v[pl.ds(g * nl, nl)]
            for j in range(nl):
                i = vec[j]
                lane = lax.rem(i, _LANES)
                start = pl.multiple_of(i - lane, _LANES)
                lanes.append(lane)
                starts.append(start)

        def fetch(b):
            return pltpu.async_copy(
                table_hbm.at[:, pl.ds(starts[b], _LANES)],
                slab_v.at[b % 2], sems.at[b % 2])

        pending = fetch(0)
        row_ids = lax.iota(jnp.int32, nl)
        for b in range(b_per_w):
            pending.wait()
            if b + 1 < b_per_w:
                pending = fetch(b + 1)
            col = jnp.full((nl,), lanes[b], jnp.int32)
            for g in range(embed // nl):
                vals = plsc.load_gather(
                    slab_v.at[b % 2], [row_ids + g * nl, col])
                rows_v[b, pl.ds(g * nl, nl)] = vals
        pltpu.async_copy(rows_v, out_hbm.at[pl.ds(base, b_per_w)], osem).wait()

    return gather_kernel(table_t, idx)


_BLOCK_N = 2048


def _score_t_tc(users_emb, items_t):
    batch, embed = users_emb.shape
    n = items_t.shape[1]

    def body(u_ref, it_ref, o_ref):
        s = lax.dot_general(
            it_ref[...], u_ref[...], (((0,), (1,)), ((), ())),
            preferred_element_type=jnp.float32,
        )
        o_ref[...] = 0.5 * jnp.tanh(0.5 * s) + 0.5

    return pl.pallas_call(
        body,
        grid=(pl.cdiv(n, _BLOCK_N),),
        in_specs=[
            pl.BlockSpec((batch, embed), lambda j: (0, 0)),
            pl.BlockSpec((embed, _BLOCK_N), lambda j: (0, j)),
        ],
        out_specs=pl.BlockSpec((_BLOCK_N, batch), lambda j: (j, 0)),
        out_shape=jax.ShapeDtypeStruct((n, batch), jnp.float32),
    )(users_emb, items_t)


def kernel(users, embedding_user_weight, embedding_item_weight):
    idx = users.astype(jnp.int32)
    users_emb = _gather_rows_sc(embedding_user_weight.T, idx)
    out_t = _score_t_tc(users_emb, embedding_item_weight.T)
    return out_t.T


# speedup vs baseline: 6.8845x; 1.1151x over previous
import functools

import jax
import jax.numpy as jnp
from jax import lax
from jax.experimental import pallas as pl
from jax.experimental.pallas import tpu as pltpu
from jax.experimental.pallas import tpu_sc as plsc

_LANES = 128


def _gather_rows_sc(table_t, idx):
    embed, _ = table_t.shape
    batch = idx.shape[0]
    info = plsc.get_sparse_core_info()
    nc, ns, nl = info.num_cores, info.num_subcores, info.num_lanes
    nw = nc * ns
    b_per_w = batch // nw
    mesh = plsc.VectorSubcoreMesh(core_axis_name="c", subcore_axis_name="s")

    @functools.partial(
        pl.kernel,
        mesh=mesh,
        compiler_params=pltpu.CompilerParams(needs_layout_passes=False),
        out_type=jax.ShapeDtypeStruct((batch, embed), jnp.float32),
        scratch_types=[
            pltpu.VMEM((b_per_w,), jnp.int32),
            pltpu.VMEM((8, embed, _LANES), jnp.float32),
            pltpu.VMEM((b_per_w, embed), jnp.float32),
            pltpu.SemaphoreType.DMA((8,)),
            pltpu.SemaphoreType.DMA,
        ],
    )
    def gather_kernel(table_hbm, idx_hbm, out_hbm, idx_v, slab_v, rows_v,
                      sems, osem):
        nbuf = 8
        wid = lax.axis_index("s") * nc + lax.axis_index("c")
        base = wid * b_per_w
        pltpu.sync_copy(idx_hbm.at[pl.ds(base, b_per_w)], idx_v)
        lanes, starts = [], []
        for g in range(b_per_w // nl):
            vec = idx_v[pl.ds(g * nl, nl)]
            for j in range(nl):
                i = vec[j]
                lane = lax.rem(i, _LANES)
                start = pl.multiple_of(i - lane, _LANES)
                lanes.append(lane)
                starts.append(start)

        def fetch(b):
            return pltpu.async_copy(
                table_hbm.at[:, pl.ds(starts[b], _LANES)],
                slab_v.at[b % nbuf], sems.at[b % nbuf])

        pend = [fetch(b) for b in range(nbuf)]
        row_ids = lax.iota(jnp.int32, nl)
        for b in range(b_per_w):
            pend[b % nbuf].wait()
            col = jnp.full((nl,), lanes[b], jnp.int32)
            for g in range(embed // nl):
                vals = plsc.load_gather(
                    slab_v.at[b % nbuf], [row_ids + g * nl, col])
                rows_v[b, pl.ds(g * nl, nl)] = vals
            if b + nbuf < b_per_w:
                pend[b % nbuf] = fetch(b + nbuf)
        pltpu.async_copy(rows_v, out_hbm.at[pl.ds(base, b_per_w)], osem).wait()

    return gather_kernel(table_t, idx)


_BLOCK_N = 2048


def _score_t_tc(users_emb, items_t):
    batch, embed = users_emb.shape
    n = items_t.shape[1]

    def body(u_ref, it_ref, o_ref):
        s = lax.dot_general(
            it_ref[...], u_ref[...], (((0,), (1,)), ((), ())),
            preferred_element_type=jnp.float32,
        )
        o_ref[...] = 0.5 * jnp.tanh(0.5 * s) + 0.5

    return pl.pallas_call(
        body,
        grid=(pl.cdiv(n, _BLOCK_N),),
        in_specs=[
            pl.BlockSpec((batch, embed), lambda j: (0, 0)),
            pl.BlockSpec((embed, _BLOCK_N), lambda j: (0, j)),
        ],
        out_specs=pl.BlockSpec((_BLOCK_N, batch), lambda j: (j, 0)),
        out_shape=jax.ShapeDtypeStruct((n, batch), jnp.float32),
    )(users_emb, items_t)


def kernel(users, embedding_user_weight, embedding_item_weight):
    idx = users.astype(jnp.int32)
    users_emb = _gather_rows_sc(embedding_user_weight.T, idx)
    out_t = _score_t_tc(users_emb, embedding_item_weight.T)
    return out_t.T
